# baseline (speedup 1.0000x reference)
import functools

import jax
import jax.numpy as jnp
from jax import lax
from jax.experimental import pallas as pl
from jax.experimental.pallas import tpu as pltpu
from jax.experimental.pallas import tpu_sc as plsc

_B, _S, _D = 4, 4096, 1024
_NC, _NS = 2, 16
_NW = _NC * _NS
_ROWS_W = _S // _NW
_CH = 16
_NCHUNK = _ROWS_W // _CH
_NVEC = _CH * _D // 16
_CPR = _D // 16
_NSLOT = 7
_LA = 5

_LOADS = []
for _c in range(_NCHUNK):
    _LOADS.append(("p", _c, 0))
    for _b in range(_B):
        _LOADS.append(("x", _c, _b))
_NL = len(_LOADS)


def _item_of(c, b):
    return c * _B + b


def _xload_k(c, b):
    return 5 * c + 1 + b


def _pload_k(c):
    return 5 * c


def _sc_add(x_hbm, pos_hbm, out_hbm, ring, lsems, ssems):
    wid = lax.axis_index("s") * _NC + lax.axis_index("c")
    row0 = wid * _ROWS_W

    n_items = _NCHUNK * _B
    load_h = [None] * _NL
    store_h = [None] * n_items
    store_waited = [False] * n_items

    def issue_load(k):
        kind, c, b = _LOADS[k]
        slot = k % _NSLOT
        prev = k - _NSLOT
        if prev >= 0 and _LOADS[prev][0] == "x":
            t_prev = _item_of(_LOADS[prev][1], _LOADS[prev][2])
            store_h[t_prev].wait()
            store_waited[t_prev] = True
        if kind == "x":
            src = x_hbm.at[b, pl.ds(row0 + c * _CH, _CH)]
        else:
            src = pos_hbm.at[pl.ds(row0 + c * _CH, _CH)]
        load_h[k] = pltpu.async_copy(src, ring.at[slot], lsems.at[slot])

    def can_issue(k, t):
        prev = k - _NSLOT
        if prev < 0:
            return True
        kind, c, b = _LOADS[prev]
        if kind == "x":
            return _item_of(c, b) < t
        return _item_of(c, _B - 1) < t

    next_k = 0
    for t in range(n_items):
        c, b = divmod(t, _B)
        target = min(_xload_k(c, b) + _LA, _NL - 1)
        while next_k <= target and can_issue(next_k, t):
            issue_load(next_k)
            next_k += 1

        if b == 0:
            load_h[_pload_k(c)].wait()
        load_h[_xload_k(c, b)].wait()

        xslot = _xload_k(c, b) % _NSLOT
        pslot = _pload_k(c) % _NSLOT
        xb = ring.at[xslot]
        pb = ring.at[pslot]

        @plsc.parallel_loop(0, _NVEC, unroll=8)
        def add_body(j):
            r = j >> 6
            cc = (j & (_CPR - 1)) * 16
            plsc.addupdate(xb.at[r, pl.ds(cc, 16)], pb[r, pl.ds(cc, 16)])

        store_h[t] = pltpu.async_copy(
            ring.at[xslot], out_hbm.at[b, pl.ds(row0 + c * _CH, _CH)],
            ssems.at[xslot])

    for t in range(n_items):
        if not store_waited[t]:
            store_h[t].wait()


_mesh = plsc.VectorSubcoreMesh(core_axis_name="c", subcore_axis_name="s")

_call = functools.partial(
    pl.kernel,
    out_type=jax.ShapeDtypeStruct((_B, _S, _D), jnp.float32),
    mesh=_mesh,
    scratch_types=[
        pltpu.VMEM((_NSLOT, _CH, _D), jnp.float32),
        pltpu.SemaphoreType.DMA((_NSLOT,)),
        pltpu.SemaphoreType.DMA((_NSLOT,)),
    ],
)(_sc_add)


@jax.jit
def kernel(x, pos_table):
    return _call(x, pos_table)
